# baseline (speedup 1.0000x reference)
import functools

import jax
import jax.numpy as jnp
from jax import lax
from jax.experimental import pallas as pl
from jax.experimental.pallas import tpu as pltpu
from jax.experimental.pallas import tpu_sc as plsc

N = 10000
E = 320000
D = 128
H = 64
L = 100
G = 16

NW = 32
RPW = 320
NP = NW * RPW
C2 = 128
NCHG = E // C2
CH = 2560
GSZ = 64
EBLK = 6400

_f32 = jnp.float32
_bf16 = jnp.bfloat16


def _sc_mesh():
    return plsc.VectorSubcoreMesh(core_axis_name="c", subcore_axis_name="s")


def _gather2(x, src, dst):
    @functools.partial(
        pl.kernel,
        out_type=[jax.ShapeDtypeStruct((E, 128), _f32),
                  jax.ShapeDtypeStruct((E, 128), _f32)],
        mesh=_sc_mesh(),
        compiler_params=pltpu.CompilerParams(needs_layout_passes=False),
        scratch_types=[
            pltpu.VMEM((C2,), jnp.int32),
            pltpu.VMEM((C2,), jnp.int32),
            pltpu.VMEM((C2, 128), _f32),
            pltpu.VMEM((C2, 128), _f32),
            pltpu.SemaphoreType.DMA,
        ],
    )
    def k(x_h, src_h, dst_h, xi_h, xj_h, dbuf, sbuf, ibuf, jbuf, sem):
        wid = lax.axis_index("s") * 2 + lax.axis_index("c")
        nch = (NCHG - 1 - wid) // NW + 1

        def chunk(i, carry):
            base = (wid + i * NW) * C2
            pltpu.sync_copy(dst_h.at[pl.ds(base, C2)], dbuf)
            pltpu.sync_copy(src_h.at[pl.ds(base, C2)], sbuf)
            c1 = pltpu.async_copy(x_h.at[dbuf], ibuf, sem)
            c2 = pltpu.async_copy(x_h.at[sbuf], jbuf, sem)
            c1.wait()
            c2.wait()
            pltpu.sync_copy(ibuf, xi_h.at[pl.ds(base, C2)])
            pltpu.sync_copy(jbuf, xj_h.at[pl.ds(base, C2)])
            return carry

        lax.fori_loop(0, nch, chunk, 0)

    return k(x, src, dst)


def _emlp_body(xi_ref, xj_ref, w0_ref, b0_ref, w1_ref, b1_ref, w2_ref, b2_ref,
               out_ref):
    xi = xi_ref[...]
    xj = xj_ref[...]
    cat = jnp.concatenate([xi, xj - xi], axis=1).astype(_bf16)
    t = jnp.dot(cat, w0_ref[...], preferred_element_type=_f32) + b0_ref[...]
    t = jnp.maximum(t, 0.0).astype(_bf16)
    t = jnp.dot(t, w1_ref[...], preferred_element_type=_f32) + b1_ref[...]
    t = jnp.maximum(t, 0.0).astype(_bf16)
    out_ref[...] = jnp.dot(t, w2_ref[...], preferred_element_type=_f32) + b2_ref[...]


def _edge_mlp(xi, xj, w0b, b0, w1b, b1, w2b, b2p):
    nblk = E // EBLK
    return pl.pallas_call(
        _emlp_body,
        grid=(nblk,),
        in_specs=[
            pl.BlockSpec((EBLK, 128), lambda i: (i, 0)),
            pl.BlockSpec((EBLK, 128), lambda i: (i, 0)),
            pl.BlockSpec((256, H), lambda i: (0, 0)),
            pl.BlockSpec((1, H), lambda i: (0, 0)),
            pl.BlockSpec((H, H), lambda i: (0, 0)),
            pl.BlockSpec((1, H), lambda i: (0, 0)),
            pl.BlockSpec((H, 128), lambda i: (0, 0)),
            pl.BlockSpec((1, 128), lambda i: (0, 0)),
        ],
        out_specs=pl.BlockSpec((EBLK, 128), lambda i: (i, 0)),
        out_shape=jax.ShapeDtypeStruct((E, 128), _f32),
    )(xi, xj, w0b, b0.reshape(1, H), w1b, b1.reshape(1, H), w2b,
      b2p.reshape(1, 128))


def _scattermax(msg, dst):
    @functools.partial(
        pl.kernel,
        out_type=jax.ShapeDtypeStruct((NP, 128), _f32),
        mesh=_sc_mesh(),
        compiler_params=pltpu.CompilerParams(needs_layout_passes=False),
        scratch_types=[
            pltpu.VMEM((CH,), jnp.int32),
            pltpu.VMEM((CH + GSZ + 16,), jnp.int32),
            pltpu.VMEM((GSZ,), jnp.int32),
            pltpu.VMEM((GSZ, 128), _f32),
            pltpu.VMEM((RPW + 1, 128), _f32),
            pltpu.SemaphoreType.DMA,
        ],
    )
    def k(msg_h, dst_h, out_h, dbuf, lbuf, gidx, rows, acc, sem):
        wid = lax.axis_index("s") * 2 + lax.axis_index("c")
        lo = wid * RPW
        hi = lo + RPW
        zero16 = jnp.zeros((16,), _f32)
        iota16 = lax.broadcasted_iota(jnp.int32, (16,), 0)
        dummy = jnp.full((16,), RPW, jnp.int32)

        def zrow(i, carry):
            for cb in range(8):
                acc[i, pl.ds(cb * 16, 16)] = zero16
            return carry

        lax.fori_loop(0, RPW + 1, zrow, 0)

        def chunk(c, carry):
            cbase = c * CH
            pltpu.sync_copy(dst_h.at[pl.ds(cbase, CH)], dbuf)

            def scan(i, cnt):
                d = dbuf[pl.ds(i * 16, 16)]
                m = (d >= lo) & (d < hi)
                eid = cbase + i * 16 + iota16
                packed = (eid << 9) | ((d - lo) & 511)
                ps = plsc.cumsum(m.astype(jnp.int32))
                plsc.store_scatter(lbuf, [cnt + ps - 1], packed, mask=m)
                return cnt + ps[15]

            cnt = lax.fori_loop(0, CH // 16, scan, 0)
            for t in range(GSZ // 16):
                lbuf[pl.ds(cnt + t * 16, 16)] = dummy
            ng = (cnt + (GSZ - 1)) // GSZ

            def group(g, carry):
                gb = g * GSZ
                for t in range(GSZ // 16):
                    gidx[pl.ds(t * 16, 16)] = lax.shift_right_logical(
                        lbuf[pl.ds(gb + t * 16, 16)], 9)
                pltpu.async_copy(msg_h.at[gidx], rows, sem).wait()

                def edge(j, carry2):
                    pv = lbuf[pl.ds(gb + j, 16)]
                    p = pv[0]
                    r = p & 511
                    for cb in range(7):
                        s = pl.ds(cb * 16, 16)
                        acc[r, s] = jnp.maximum(acc[r, s], rows[j, s])
                    return carry2

                lax.fori_loop(0, GSZ, edge, 0)
                return carry

            lax.fori_loop(0, ng, group, 0)
            return carry

        lax.fori_loop(0, E // CH, chunk, 0)
        pltpu.sync_copy(acc.at[pl.ds(0, RPW)], out_h.at[pl.ds(lo, RPW)])

    return k(msg, dst)


def _pool_body(h_ref, b_ref, ones_ref, w0_ref, b0_ref, w1_ref, b1_ref,
               w2_ref, b2_ref, out_ref):
    hv = h_ref[...][:N, :]
    bv = b_ref[...]
    onehot = (bv == lax.broadcasted_iota(jnp.int32, (1, G), 1)).astype(_f32)
    addp = lax.dot_general(onehot, hv, (((0,), (0,)), ((), ())),
                           preferred_element_type=_f32,
                           precision=lax.Precision.HIGHEST)
    cnt = lax.dot_general(onehot, ones_ref[...], (((0,), (0,)), ((), ())),
                          preferred_element_type=_f32,
                          precision=lax.Precision.HIGHEST)
    meanp = addp / jnp.maximum(cnt, 1.0)
    cols = [
        jnp.max(jnp.where(bv == g, hv, -1.0), axis=0, keepdims=True)
        for g in range(G)
    ]
    maxp = jnp.maximum(jnp.concatenate(cols, axis=0), 0.0)
    pooled = jnp.concatenate([addp, meanp, maxp], axis=1).astype(_bf16)
    t = jnp.maximum(
        jnp.dot(pooled, w0_ref[...], preferred_element_type=_f32)
        + b0_ref[...], 0.0).astype(_bf16)
    t = jnp.maximum(
        jnp.dot(t, w1_ref[...], preferred_element_type=_f32)
        + b1_ref[...], 0.0).astype(_bf16)
    out_ref[...] = jnp.dot(t, w2_ref[...], preferred_element_type=_f32) + b2_ref[...]


def _pool(h2, batch, w0b, b0, w1b, b1, w2b, b2):
    return pl.pallas_call(
        _pool_body,
        out_shape=jax.ShapeDtypeStruct((G, 1), _f32),
    )(h2, batch.reshape(N, 1), jnp.ones((N, 1), _f32), w0b,
      b0.reshape(1, L), w1b, b1.reshape(1, L), w2b, b2.reshape(1, 1))


def _edge_layer(h, src, dst, w0b, b0, w1b, b1, w2b, b2p):
    xi, xj = _gather2(h, src, dst)
    msg = _edge_mlp(xi, xj, w0b, b0, w1b, b1, w2b, b2p)
    return _scattermax(msg, dst)


def kernel(x, pos, u, l0_w0, l0_b0, l0_w1, l0_b1, l0_w2, l0_b2,
           l1_w0, l1_b0, l1_w1, l1_b1, l1_w2, l1_b2,
           lin_w0, lin_b0, lin_w1, lin_b1, lin_w2, lin_b2,
           edge_index, batch):
    src = edge_index[0]
    dst = edge_index[1]

    w0b0 = l0_w0.astype(_bf16)
    w2p0 = jnp.pad(l0_w2, ((0, 0), (0, 128 - L))).astype(_bf16)
    b2p0 = jnp.pad(l0_b2, (0, 128 - L))

    w0p1 = jnp.zeros((256, H), _f32)
    w0p1 = w0p1.at[0:L].set(l1_w0[0:L])
    w0p1 = w0p1.at[128:128 + L].set(l1_w0[L:2 * L])
    w0b1 = w0p1.astype(_bf16)
    w2p1 = jnp.pad(l1_w2, ((0, 0), (0, 128 - L))).astype(_bf16)
    b2p1 = jnp.pad(l1_b2, (0, 128 - L))

    w0p = jnp.zeros((384, L), _f32)
    w0p = w0p.at[0:L].set(lin_w0[0:L])
    w0p = w0p.at[128:128 + L].set(lin_w0[L:2 * L])
    w0p = w0p.at[256:256 + L].set(lin_w0[2 * L:3 * L])

    h1 = _edge_layer(x, src, dst, w0b0, l0_b0, l0_w1.astype(_bf16), l0_b1,
                     w2p0, b2p0)
    h2 = _edge_layer(h1, src, dst, w0b1, l1_b0, l1_w1.astype(_bf16), l1_b1,
                     w2p1, b2p1)
    return _pool(h2, batch, w0p.astype(_bf16), lin_b0,
                 lin_w1.astype(_bf16), lin_b1, lin_w2.astype(_bf16), lin_b2)
